# baseline (speedup 1.0000x reference)
import jax
import jax.numpy as jnp
from jax import lax
from jax.experimental import pallas as pl
from jax.experimental.pallas import tpu as pltpu
from jax.experimental.pallas import tpu_sc as plsc

L = 16
NC = 2
NS = 16
NW = NC * NS

B, R, C = 8, 20000, 128
G = C // L
WPB = NW // B
SPAN = 5120
CHUNK = 128
NBUF = 4
NCHUNK = SPAN // CHUNK
GPC = CHUNK // L

BIG = 0x7F000000


def _worker_span(q):
  start = q * 5000 + 8 * (q % 2)
  start = jnp.where(q == WPB - 1, R - SPAN, start)
  return pl.multiple_of(start, 16)


def _body(sim, mcols, mvals, buf0, buf1, buf2, buf3, ovbuf, oibuf, tv, ti,
          table, mcbuf, mvbuf, pvbuf, prbuf, pvb, prb, pv_sh, pr_sh,
          sem0, sem1, sem2, sem3):
  core = lax.axis_index("c")
  sub = lax.axis_index("s")
  b = core * (NS // WPB) + sub // WPB
  q = sub % WPB
  start = _worker_span(q)

  iota = lax.broadcasted_iota(jnp.int32, (L,), 0)
  iota16 = iota * L
  ones = jnp.full((L,), 1, jnp.int32)
  idx_cf = [(iota + L * g).astype(jnp.float32) for g in range(G)]
  idx_ci = [iota + L * g for g in range(G)]
  onehot = [iota == r for r in range(L)]
  bigf = jnp.full((L,), 1e9, jnp.float32)
  bigv = jnp.full((L,), BIG, jnp.int32)
  padv = jnp.full((L,), -1.0, jnp.float32)
  padif = jnp.full((L,), float(C), jnp.float32)

  bufs = (buf0, buf1, buf2, buf3)
  sems = (sem0, sem1, sem2, sem3)

  def _src(ch):
    row0 = pl.multiple_of(start + ch * CHUNK, 16)
    return sim.at[b, pl.ds(row0, CHUNK)]

  for p in range(NBUF - 1):
    pltpu.async_copy(_src(p), bufs[p], sems[p])

  def chunk_compute(buf, row0, carry):
    def rbody(rloc, cr):
      cmax, crow = cr
      rowvec = jnp.full((L,), row0 + rloc, jnp.int32)
      v = [buf[rloc, pl.ds(L * g, L)] for g in range(G)]
      ncmax, ncrow = [], []
      for g in range(G):
        m = v[g] > cmax[g]
        ncmax.append(jnp.where(m, v[g], cmax[g]))
        ncrow.append(jnp.where(m, rowvec, crow[g]))
      a = list(v)
      n = G
      while n > 1:
        a = [jnp.maximum(a[p], a[p + 1]) for p in range(0, n, 2)]
        n //= 2
      mx = a[0]
      c = [jnp.where(v[g] == mx, idx_cf[g], bigf) for g in range(G)]
      n = G
      while n > 1:
        c = [jnp.minimum(c[p], c[p + 1]) for p in range(0, n, 2)]
        n //= 2
      tv[pl.ds(rloc * L, L)] = mx
      ti[pl.ds(rloc * L, L)] = c[0]
      return tuple(ncmax), tuple(ncrow)

    carry = plsc.parallel_loop(0, CHUNK, 1, unroll=2, carry=carry)(rbody)

    def tbody(t, _):
      base = jnp.full((L,), t * (L * L), jnp.int32) + iota16
      idxv = base
      mxv = plsc.load_gather(tv, [idxv])
      for _j in range(1, L):
        idxv = idxv + ones
        mxv = jnp.maximum(mxv, plsc.load_gather(tv, [idxv]))
      idxv = base
      amv = jnp.where(plsc.load_gather(tv, [idxv]) == mxv,
                      plsc.load_gather(ti, [idxv]), bigf)
      for _j in range(1, L):
        idxv = idxv + ones
        amv = jnp.minimum(
            amv,
            jnp.where(plsc.load_gather(tv, [idxv]) == mxv,
                      plsc.load_gather(ti, [idxv]), bigf))
      padw = mxv < padv
      mxv = jnp.where(padw, padv, mxv)
      amv = jnp.where(padw, padif, amv)
      off = row0 - start + t * L
      ovbuf[pl.ds(off, L)] = mxv
      oibuf[pl.ds(off, L)] = amv.astype(jnp.int32)
      return 0

    plsc.parallel_loop(0, GPC, 1, unroll=1, carry=jnp.int32(0))(tbody)
    return carry

  def outer_body(i, carry):
    for bi in range(NBUF):
      ch = NBUF * i + bi
      pltpu.make_async_copy(_src(ch), bufs[bi], sems[bi]).wait()

      @pl.when(ch + NBUF - 1 < NCHUNK)
      def _():
        nb = (bi + NBUF - 1) % NBUF
        pltpu.async_copy(_src(ch + NBUF - 1), bufs[nb], sems[nb])

      row0 = pl.multiple_of(start + ch * CHUNK, 16)
      carry = chunk_compute(bufs[bi], row0, carry)
    return carry

  cmax0 = tuple(jnp.full((L,), -jnp.inf, jnp.float32) for _ in range(G))
  crow0 = tuple(jnp.zeros((L,), jnp.int32) for _ in range(G))
  cmax, crow = lax.fori_loop(0, NCHUNK // NBUF, outer_body, (cmax0, crow0))

  for g in range(G):
    pvbuf[pl.ds(L * g, L)] = cmax[g]
    prbuf[pl.ds(L * g, L)] = crow[g]
  sofs = pl.multiple_of(sub * C, 16)
  pltpu.sync_copy(pvbuf, pv_sh.at[pl.ds(sofs, C)])
  pltpu.sync_copy(prbuf, pr_sh.at[pl.ds(sofs, C)])
  plsc.subcore_barrier()

  bofs = pl.multiple_of((sub // WPB) * WPB * C, 16)
  pltpu.sync_copy(pv_sh.at[pl.ds(bofs, WPB * C)], pvb)
  pltpu.sync_copy(pr_sh.at[pl.ds(bofs, WPB * C)], prb)
  mrow = []
  for g in range(G):
    cur = pvb[pl.ds(L * g, L)]
    curr = prb[pl.ds(L * g, L)]
    for k in range(1, WPB):
      vk = pvb[pl.ds(k * C + L * g, L)]
      rk = prb[pl.ds(k * C + L * g, L)]
      m = vk > cur
      cur = jnp.where(m, vk, cur)
      curr = jnp.where(m, rk, curr)
    mrow.append(curr)

  def init_body(i, _):
    table[pl.ds(i * L, L)] = bigv
    return 0
  plsc.parallel_loop(0, SPAN // L, 1, unroll=2, carry=jnp.int32(0))(init_body)

  startv = jnp.full((L,), start, jnp.int32)
  pad_idx = jnp.zeros((L,), jnp.int32) - startv
  pad_in = (pad_idx >= 0) & (pad_idx < SPAN)
  plsc.store_scatter(table, [pad_idx], jnp.full((L,), C, jnp.int32),
                     mask=pad_in & onehot[0])
  for g in reversed(range(G)):
    rcl = mrow[g] - startv
    inr = (rcl >= 0) & (rcl < SPAN)
    for lane in reversed(range(L)):
      plsc.store_scatter(table, [rcl], idx_ci[g], mask=inr & onehot[lane])

  one = jnp.full((L,), 1, jnp.int32)
  neg1 = jnp.full((L,), -1, jnp.int32)
  neg2 = jnp.full((L,), -2, jnp.int32)

  def comb_body(i, _):
    sl = pl.ds(i * L, L)
    f = table[sl]
    forced = f < bigv
    rm = ovbuf[sl]
    cls = jnp.where(rm >= jnp.float32(0.5), one,
                    jnp.where(rm >= jnp.float32(0.4), neg2, neg1))
    mcbuf[sl] = jnp.where(forced, f, oibuf[sl])
    mvbuf[sl] = jnp.where(forced, one, cls)
    return 0
  plsc.parallel_loop(0, SPAN // L, 1, unroll=2, carry=jnp.int32(0))(comb_body)

  rofs = pl.multiple_of(b * R + start, 16)
  pltpu.sync_copy(mcbuf, mcols.at[pl.ds(rofs, SPAN)])
  pltpu.sync_copy(mvbuf, mvals.at[pl.ds(rofs, SPAN)])


def kernel(similarity_matrix):
  assert similarity_matrix.shape == (B, R, C)
  mesh = plsc.VectorSubcoreMesh(core_axis_name="c", subcore_axis_name="s")
  params = pltpu.CompilerParams(needs_layout_passes=False)

  mcols, mvals = pl.kernel(
      _body,
      out_type=[
          jax.ShapeDtypeStruct((B * R,), jnp.int32),
          jax.ShapeDtypeStruct((B * R,), jnp.int32),
      ],
      mesh=mesh,
      scratch_types=[
          pltpu.VMEM((CHUNK, C), jnp.float32),
          pltpu.VMEM((CHUNK, C), jnp.float32),
          pltpu.VMEM((CHUNK, C), jnp.float32),
          pltpu.VMEM((CHUNK, C), jnp.float32),
          pltpu.VMEM((SPAN,), jnp.float32),
          pltpu.VMEM((SPAN,), jnp.int32),
          pltpu.VMEM((CHUNK * L,), jnp.float32),
          pltpu.VMEM((CHUNK * L,), jnp.float32),
          pltpu.VMEM((SPAN,), jnp.int32),
          pltpu.VMEM((SPAN,), jnp.int32),
          pltpu.VMEM((SPAN,), jnp.int32),
          pltpu.VMEM((C,), jnp.float32),
          pltpu.VMEM((C,), jnp.int32),
          pltpu.VMEM((WPB * C,), jnp.float32),
          pltpu.VMEM((WPB * C,), jnp.int32),
          pltpu.VMEM_SHARED((NS * C,), jnp.float32),
          pltpu.VMEM_SHARED((NS * C,), jnp.int32),
          pltpu.SemaphoreType.DMA,
          pltpu.SemaphoreType.DMA,
          pltpu.SemaphoreType.DMA,
          pltpu.SemaphoreType.DMA,
      ],
      compiler_params=params,
  )(similarity_matrix)

  return mcols.reshape(B, R), mvals.reshape(B, R)


# speedup vs baseline: 1.0576x; 1.0576x over previous
import jax
import jax.numpy as jnp
from jax import lax
from jax.experimental import pallas as pl
from jax.experimental.pallas import tpu as pltpu
from jax.experimental.pallas import tpu_sc as plsc

L = 16
NC = 2
NS = 16
NW = NC * NS

B, R, C = 8, 20000, 128
G = C // L
WPB = NW // B
SPAN = 5120
CHUNK = 256
NBUF = 2
NCHUNK = SPAN // CHUNK
GPC = CHUNK // L

BIG = 0x7F000000


def _worker_span(q):
  start = q * 5000 + 8 * (q % 2)
  start = jnp.where(q == WPB - 1, R - SPAN, start)
  return pl.multiple_of(start, 16)


def _body(sim, mcols, mvals, buf0, buf1, ovbuf, oibuf, tv, ti,
          table, mcbuf, mvbuf, pvbuf, prbuf, pvb, prb, pv_sh, pr_sh,
          sem0, sem1):
  core = lax.axis_index("c")
  sub = lax.axis_index("s")
  b = core * (NS // WPB) + sub // WPB
  q = sub % WPB
  start = _worker_span(q)

  iota = lax.broadcasted_iota(jnp.int32, (L,), 0)
  iota16 = iota * L
  ones = jnp.full((L,), 1, jnp.int32)
  idx_cf = [(iota + L * g).astype(jnp.float32) for g in range(G)]
  idx_ci = [iota + L * g for g in range(G)]
  onehot = [iota == r for r in range(L)]
  bigf = jnp.full((L,), 1e9, jnp.float32)
  bigv = jnp.full((L,), BIG, jnp.int32)
  padv = jnp.full((L,), -1.0, jnp.float32)
  padif = jnp.full((L,), float(C), jnp.float32)

  bufs = (buf0, buf1)
  sems = (sem0, sem1)

  def _src(ch):
    row0 = pl.multiple_of(start + ch * CHUNK, 16)
    return sim.at[b, pl.ds(row0, CHUNK)]

  for p in range(NBUF - 1):
    pltpu.async_copy(_src(p), bufs[p], sems[p])

  def chunk_compute(buf, row0, carry):
    def rbody(rloc, cr):
      cmax, crow = cr
      rowvec = jnp.full((L,), row0 + rloc, jnp.int32)
      v = [buf[rloc, pl.ds(L * g, L)] for g in range(G)]
      ncmax, ncrow = [], []
      for g in range(G):
        m = v[g] > cmax[g]
        ncmax.append(jnp.where(m, v[g], cmax[g]))
        ncrow.append(jnp.where(m, rowvec, crow[g]))
      a = list(v)
      n = G
      while n > 1:
        a = [jnp.maximum(a[p], a[p + 1]) for p in range(0, n, 2)]
        n //= 2
      mx = a[0]
      c = [jnp.where(v[g] == mx, idx_cf[g], bigf) for g in range(G)]
      n = G
      while n > 1:
        c = [jnp.minimum(c[p], c[p + 1]) for p in range(0, n, 2)]
        n //= 2
      tv[pl.ds(rloc * L, L)] = mx
      ti[pl.ds(rloc * L, L)] = c[0]
      return tuple(ncmax), tuple(ncrow)

    carry = plsc.parallel_loop(0, CHUNK, 1, unroll=2, carry=carry)(rbody)

    def tbody(t, _):
      base = jnp.full((L,), t * (L * L), jnp.int32) + iota16
      idxv = base
      mxv = plsc.load_gather(tv, [idxv])
      for _j in range(1, L):
        idxv = idxv + ones
        mxv = jnp.maximum(mxv, plsc.load_gather(tv, [idxv]))
      idxv = base
      amv = jnp.where(plsc.load_gather(tv, [idxv]) == mxv,
                      plsc.load_gather(ti, [idxv]), bigf)
      for _j in range(1, L):
        idxv = idxv + ones
        amv = jnp.minimum(
            amv,
            jnp.where(plsc.load_gather(tv, [idxv]) == mxv,
                      plsc.load_gather(ti, [idxv]), bigf))
      padw = mxv < padv
      mxv = jnp.where(padw, padv, mxv)
      amv = jnp.where(padw, padif, amv)
      off = row0 - start + t * L
      ovbuf[pl.ds(off, L)] = mxv
      oibuf[pl.ds(off, L)] = amv.astype(jnp.int32)
      return 0

    plsc.parallel_loop(0, GPC, 1, unroll=1, carry=jnp.int32(0))(tbody)
    return carry

  def outer_body(i, carry):
    for bi in range(NBUF):
      ch = NBUF * i + bi
      pltpu.make_async_copy(_src(ch), bufs[bi], sems[bi]).wait()

      @pl.when(ch + NBUF - 1 < NCHUNK)
      def _():
        nb = (bi + NBUF - 1) % NBUF
        pltpu.async_copy(_src(ch + NBUF - 1), bufs[nb], sems[nb])

      row0 = pl.multiple_of(start + ch * CHUNK, 16)
      carry = chunk_compute(bufs[bi], row0, carry)
    return carry

  cmax0 = tuple(jnp.full((L,), -jnp.inf, jnp.float32) for _ in range(G))
  crow0 = tuple(jnp.zeros((L,), jnp.int32) for _ in range(G))
  cmax, crow = lax.fori_loop(0, NCHUNK // NBUF, outer_body, (cmax0, crow0))

  for g in range(G):
    pvbuf[pl.ds(L * g, L)] = cmax[g]
    prbuf[pl.ds(L * g, L)] = crow[g]
  sofs = pl.multiple_of(sub * C, 16)
  pltpu.sync_copy(pvbuf, pv_sh.at[pl.ds(sofs, C)])
  pltpu.sync_copy(prbuf, pr_sh.at[pl.ds(sofs, C)])
  plsc.subcore_barrier()

  bofs = pl.multiple_of((sub // WPB) * WPB * C, 16)
  pltpu.sync_copy(pv_sh.at[pl.ds(bofs, WPB * C)], pvb)
  pltpu.sync_copy(pr_sh.at[pl.ds(bofs, WPB * C)], prb)
  mrow = []
  for g in range(G):
    cur = pvb[pl.ds(L * g, L)]
    curr = prb[pl.ds(L * g, L)]
    for k in range(1, WPB):
      vk = pvb[pl.ds(k * C + L * g, L)]
      rk = prb[pl.ds(k * C + L * g, L)]
      m = vk > cur
      cur = jnp.where(m, vk, cur)
      curr = jnp.where(m, rk, curr)
    mrow.append(curr)

  def init_body(i, _):
    table[pl.ds(i * L, L)] = bigv
    return 0
  plsc.parallel_loop(0, SPAN // L, 1, unroll=2, carry=jnp.int32(0))(init_body)

  startv = jnp.full((L,), start, jnp.int32)
  pad_idx = jnp.zeros((L,), jnp.int32) - startv
  pad_in = (pad_idx >= 0) & (pad_idx < SPAN)
  plsc.store_scatter(table, [pad_idx], jnp.full((L,), C, jnp.int32),
                     mask=pad_in & onehot[0])
  for g in reversed(range(G)):
    rcl = mrow[g] - startv
    inr = (rcl >= 0) & (rcl < SPAN)
    for lane in reversed(range(L)):
      plsc.store_scatter(table, [rcl], idx_ci[g], mask=inr & onehot[lane])

  one = jnp.full((L,), 1, jnp.int32)
  neg1 = jnp.full((L,), -1, jnp.int32)
  neg2 = jnp.full((L,), -2, jnp.int32)

  def comb_body(i, _):
    sl = pl.ds(i * L, L)
    f = table[sl]
    forced = f < bigv
    rm = ovbuf[sl]
    cls = jnp.where(rm >= jnp.float32(0.5), one,
                    jnp.where(rm >= jnp.float32(0.4), neg2, neg1))
    mcbuf[sl] = jnp.where(forced, f, oibuf[sl])
    mvbuf[sl] = jnp.where(forced, one, cls)
    return 0
  plsc.parallel_loop(0, SPAN // L, 1, unroll=2, carry=jnp.int32(0))(comb_body)

  rofs = pl.multiple_of(b * R + start, 16)
  pltpu.sync_copy(mcbuf, mcols.at[pl.ds(rofs, SPAN)])
  pltpu.sync_copy(mvbuf, mvals.at[pl.ds(rofs, SPAN)])


def kernel(similarity_matrix):
  assert similarity_matrix.shape == (B, R, C)
  mesh = plsc.VectorSubcoreMesh(core_axis_name="c", subcore_axis_name="s")
  params = pltpu.CompilerParams(needs_layout_passes=False)

  mcols, mvals = pl.kernel(
      _body,
      out_type=[
          jax.ShapeDtypeStruct((B * R,), jnp.int32),
          jax.ShapeDtypeStruct((B * R,), jnp.int32),
      ],
      mesh=mesh,
      scratch_types=[
          pltpu.VMEM((CHUNK, C), jnp.float32),
          pltpu.VMEM((CHUNK, C), jnp.float32),
          pltpu.VMEM((SPAN,), jnp.float32),
          pltpu.VMEM((SPAN,), jnp.int32),
          pltpu.VMEM((CHUNK * L,), jnp.float32),
          pltpu.VMEM((CHUNK * L,), jnp.float32),
          pltpu.VMEM((SPAN,), jnp.int32),
          pltpu.VMEM((SPAN,), jnp.int32),
          pltpu.VMEM((SPAN,), jnp.int32),
          pltpu.VMEM((C,), jnp.float32),
          pltpu.VMEM((C,), jnp.int32),
          pltpu.VMEM((WPB * C,), jnp.float32),
          pltpu.VMEM((WPB * C,), jnp.int32),
          pltpu.VMEM_SHARED((NS * C,), jnp.float32),
          pltpu.VMEM_SHARED((NS * C,), jnp.int32),
          pltpu.SemaphoreType.DMA,
          pltpu.SemaphoreType.DMA,
      ],
      compiler_params=params,
  )(similarity_matrix)

  return mcols.reshape(B, R), mvals.reshape(B, R)
